# Initial kernel scaffold; baseline (speedup 1.0000x reference)
import functools

import jax
import jax.numpy as jnp
from jax import lax
from jax.experimental import pallas as pl
from jax.experimental.pallas import tpu as pltpu
from jax.experimental.pallas import tpu_sc as plsc

NUM_CORES = 2
NUM_SUBCORES = 16
NUM_WORKERS = NUM_CORES * NUM_SUBCORES
LANES = 16

B = 16384
F = 26
FPAD = 32
D = 32
BPW = B // NUM_WORKERS
ROWS_PER_CHUNK = 4
CHUNKS = BPW // ROWS_PER_CHUNK
IDX_PER_CHUNK = ROWS_PER_CHUNK * F
PAD_PER_CHUNK = ROWS_PER_CHUNK * FPAD


def _fm_body(x2d_hbm, xp2d_hbm, w0_hbm, bias_hbm, emb_hbm, out_hbm,
             xv, xpv, w0v, eb0, eb1, bb0, bb1, outv,
             se0, se1, sb0, sb1):
  wid = lax.axis_index("s") * NUM_CORES + lax.axis_index("c")
  crow = pl.multiple_of(wid * CHUNKS, CHUNKS)
  pltpu.sync_copy(x2d_hbm.at[pl.ds(crow, CHUNKS), :], xv)
  pltpu.sync_copy(xp2d_hbm.at[pl.ds(crow, CHUNKS), :], xpv)
  pltpu.sync_copy(w0_hbm, w0v)
  w0s = w0v[0]

  ebufs = (eb0, eb1)
  bbufs = (bb0, bb1)
  esems = (se0, se1)
  bsems = (sb0, sb1)

  def issue(c, par):
    pltpu.async_copy(emb_hbm.at[xv.at[c]], ebufs[par], esems[par])
    pltpu.async_copy(bias_hbm.at[xpv.at[c]], bbufs[par], bsems[par])

  def wait(c, par):
    pltpu.make_async_copy(emb_hbm.at[xv.at[c]], ebufs[par], esems[par]).wait()
    pltpu.make_async_copy(bias_hbm.at[xpv.at[c]], bbufs[par], bsems[par]).wait()

  bias_mask = lax.iota(jnp.int32, LANES) < (F - LANES)

  def compute(c, par):
    eb = ebufs[par]
    bb = bbufs[par]
    for r in range(ROWS_PER_CHUNK):
      acc0 = jnp.zeros((LANES,), jnp.float32)
      acc1 = jnp.zeros((LANES,), jnp.float32)
      sq0 = jnp.zeros((LANES,), jnp.float32)
      sq1 = jnp.zeros((LANES,), jnp.float32)
      for f in range(F):
        row = r * F + f
        v0 = eb[row, pl.ds(0, LANES)]
        v1 = eb[row, pl.ds(LANES, LANES)]
        acc0 = acc0 + v0
        sq0 = sq0 + v0 * v0
        acc1 = acc1 + v1
        sq1 = sq1 + v1 * v1
      t = acc0 * acc0 + acc1 * acc1 - sq0 - sq1
      inter = 0.5 * jnp.sum(t)
      b0 = bb[pl.ds(r * FPAD, LANES)]
      b1 = bb[pl.ds(r * FPAD + LANES, LANES)]
      b1 = jnp.where(bias_mask, b1, 0.0)
      bias_sum = jnp.sum(b0) + jnp.sum(b1)
      outv[c * ROWS_PER_CHUNK + r] = inter + bias_sum + w0s

  issue(0, 0)
  issue(1, 1)

  @pl.loop(0, CHUNKS // 2)
  def _chunk_loop(i):
    c = i * 2
    for par in range(2):
      cc = c + par
      wait(cc, par)
      compute(cc, par)
      nxt = cc + 2

      @pl.when(nxt < CHUNKS)
      def _():
        issue(nxt, par)

  pltpu.sync_copy(outv, out_hbm.at[pl.ds(pl.multiple_of(wid * BPW, BPW), BPW)])


@jax.jit
def _fm_call(x2d, xp2d, w016, bias_flat, emb_table):
  return pl.kernel(
      _fm_body,
      out_type=jax.ShapeDtypeStruct((B,), jnp.float32),
      mesh=plsc.VectorSubcoreMesh(core_axis_name="c", subcore_axis_name="s"),
      scratch_types=[
          pltpu.VMEM((CHUNKS, IDX_PER_CHUNK), jnp.int32),
          pltpu.VMEM((CHUNKS, PAD_PER_CHUNK), jnp.int32),
          pltpu.VMEM((LANES,), jnp.float32),
          pltpu.VMEM((IDX_PER_CHUNK, D), jnp.float32),
          pltpu.VMEM((IDX_PER_CHUNK, D), jnp.float32),
          pltpu.VMEM((PAD_PER_CHUNK,), jnp.float32),
          pltpu.VMEM((PAD_PER_CHUNK,), jnp.float32),
          pltpu.VMEM((BPW,), jnp.float32),
          pltpu.SemaphoreType.DMA,
          pltpu.SemaphoreType.DMA,
          pltpu.SemaphoreType.DMA,
          pltpu.SemaphoreType.DMA,
      ],
  )(x2d, xp2d, w016, bias_flat, emb_table)


def kernel(x, w0, bias_table, emb_table):
  x = x.astype(jnp.int32)
  xpad = jnp.pad(x, ((0, 0), (0, FPAD - F)))
  x2d = x.reshape(NUM_WORKERS * CHUNKS, IDX_PER_CHUNK)
  xp2d = xpad.reshape(NUM_WORKERS * CHUNKS, PAD_PER_CHUNK)
  w016 = jnp.broadcast_to(w0.astype(jnp.float32), (LANES,))
  bias_flat = bias_table.reshape(-1)
  return _fm_call(x2d, xp2d, w016, bias_flat, emb_table)


# speedup vs baseline: 1.2100x; 1.2100x over previous
import jax
import jax.numpy as jnp
from jax import lax
from jax.experimental import pallas as pl
from jax.experimental.pallas import tpu as pltpu
from jax.experimental.pallas import tpu_sc as plsc

NUM_CORES = 2
NUM_SUBCORES = 16
NUM_WORKERS = NUM_CORES * NUM_SUBCORES
LANES = 16

B = 16384
F = 26
FPAD = 32
D = 32
BPW = B // NUM_WORKERS
ROWS_PER_CHUNK = 8
CHUNKS = BPW // ROWS_PER_CHUNK
SUBGATHERS = 2
IDX_PER_GATHER = IDX = (ROWS_PER_CHUNK // SUBGATHERS) * F
PAD_PER_GATHER = (ROWS_PER_CHUNK // SUBGATHERS) * FPAD
IDX_PER_CHUNK = ROWS_PER_CHUNK * F
PAD_PER_CHUNK = ROWS_PER_CHUNK * FPAD
IDX_ROWS = NUM_WORKERS * CHUNKS * SUBGATHERS // NUM_WORKERS


def _fm_body(x2d_hbm, xp2d_hbm, w0_hbm, bias_hbm, emb_hbm, out_hbm,
             xv, xpv, w0v, eb0, eb1, bb0, bb1, outv,
             se0, se1, sb0, sb1):
  wid = lax.axis_index("s") * NUM_CORES + lax.axis_index("c")
  crow = pl.multiple_of(wid * IDX_ROWS, IDX_ROWS)
  pltpu.sync_copy(x2d_hbm.at[pl.ds(crow, IDX_ROWS), :], xv)
  pltpu.sync_copy(xp2d_hbm.at[pl.ds(crow, IDX_ROWS), :], xpv)
  pltpu.sync_copy(w0_hbm, w0v)

  ebufs = (eb0, eb1)
  bbufs = (bb0, bb1)
  esems = (se0, se1)
  bsems = (sb0, sb1)

  def copies(c, par):
    for j in range(SUBGATHERS):
      yield pltpu.make_async_copy(
          emb_hbm.at[xv.at[c * SUBGATHERS + j]],
          ebufs[par].at[pl.ds(j * IDX_PER_GATHER, IDX_PER_GATHER)],
          esems[par])
      yield pltpu.make_async_copy(
          bias_hbm.at[xpv.at[c * SUBGATHERS + j]],
          bbufs[par].at[pl.ds(j * PAD_PER_GATHER, PAD_PER_GATHER)],
          bsems[par])

  def issue(c, par):
    for cp in copies(c, par):
      cp.start()

  def wait(c, par):
    for cp in copies(c, par):
      cp.wait()

  lane = lax.iota(jnp.int32, LANES)
  bias_mask = lane < (F - LANES)
  res_mask = lane < ROWS_PER_CHUNK

  def compute(c, par):
    eb = ebufs[par]
    bb = bbufs[par]
    res = jnp.zeros((LANES,), jnp.float32)
    for r in range(ROWS_PER_CHUNK):
      acc0 = jnp.zeros((LANES,), jnp.float32)
      acc1 = jnp.zeros((LANES,), jnp.float32)
      sq0 = jnp.zeros((LANES,), jnp.float32)
      sq1 = jnp.zeros((LANES,), jnp.float32)
      for f in range(F):
        row = r * F + f
        v0 = eb[row, pl.ds(0, LANES)]
        v1 = eb[row, pl.ds(LANES, LANES)]
        acc0 = acc0 + v0
        sq0 = sq0 + v0 * v0
        acc1 = acc1 + v1
        sq1 = sq1 + v1 * v1
      t = acc0 * acc0 + acc1 * acc1 - sq0 - sq1
      b0 = bb[pl.ds(r * FPAD, LANES)]
      b1 = bb[pl.ds(r * FPAD + LANES, LANES)]
      u = 0.5 * t + b0 + jnp.where(bias_mask, b1, 0.0)
      total = jnp.sum(u)
      res = jnp.where(lane == r, total, res)
    res = res + w0v[...]
    plsc.store_compressed(outv.at[pl.ds(c * ROWS_PER_CHUNK, LANES)],
                          res, mask=res_mask)

  issue(0, 0)
  issue(1, 1)

  @pl.loop(0, CHUNKS // 2)
  def _chunk_loop(i):
    c = i * 2
    for par in range(2):
      cc = c + par
      wait(cc, par)
      compute(cc, par)
      nxt = cc + 2

      @pl.when(nxt < CHUNKS)
      def _():
        issue(nxt, par)

  pltpu.sync_copy(outv.at[pl.ds(0, BPW)],
                  out_hbm.at[pl.ds(pl.multiple_of(wid * BPW, BPW), BPW)])


@jax.jit
def _fm_call(x2d, xp2d, w016, bias_flat, emb_table):
  return pl.kernel(
      _fm_body,
      out_type=jax.ShapeDtypeStruct((B,), jnp.float32),
      mesh=plsc.VectorSubcoreMesh(core_axis_name="c", subcore_axis_name="s"),
      compiler_params=pltpu.CompilerParams(
          needs_layout_passes=False, use_tc_tiling_on_sc=False),
      scratch_types=[
          pltpu.VMEM((IDX_ROWS, IDX_PER_GATHER), jnp.int32),
          pltpu.VMEM((IDX_ROWS, PAD_PER_GATHER), jnp.int32),
          pltpu.VMEM((LANES,), jnp.float32),
          pltpu.VMEM((IDX_PER_CHUNK, D), jnp.float32),
          pltpu.VMEM((IDX_PER_CHUNK, D), jnp.float32),
          pltpu.VMEM((PAD_PER_CHUNK,), jnp.float32),
          pltpu.VMEM((PAD_PER_CHUNK,), jnp.float32),
          pltpu.VMEM((BPW + LANES,), jnp.float32),
          pltpu.SemaphoreType.DMA,
          pltpu.SemaphoreType.DMA,
          pltpu.SemaphoreType.DMA,
          pltpu.SemaphoreType.DMA,
      ],
  )(x2d, xp2d, w016, bias_flat, emb_table)


def kernel(x, w0, bias_table, emb_table):
  x = x.astype(jnp.int32)
  xpad = jnp.pad(x, ((0, 0), (0, FPAD - F)))
  x2d = x.reshape(NUM_WORKERS * CHUNKS * SUBGATHERS, IDX_PER_GATHER)
  xp2d = xpad.reshape(NUM_WORKERS * CHUNKS * SUBGATHERS, PAD_PER_GATHER)
  w016 = jnp.broadcast_to(w0.astype(jnp.float32), (LANES,))
  bias_flat = bias_table.reshape(-1)
  return _fm_call(x2d, xp2d, w016, bias_flat, emb_table)


# speedup vs baseline: 1.2114x; 1.0011x over previous
import jax
import jax.numpy as jnp
from jax import lax
from jax.experimental import pallas as pl
from jax.experimental.pallas import tpu as pltpu
from jax.experimental.pallas import tpu_sc as plsc

NUM_CORES = 2
NUM_SUBCORES = 16
NUM_WORKERS = NUM_CORES * NUM_SUBCORES
LANES = 16

B = 16384
F = 26
FPAD = 32
D = 32
NUM_FEATURES = 1000000
BPW = B // NUM_WORKERS
ROWS_PER_CHUNK = 8
CHUNKS = BPW // ROWS_PER_CHUNK
SUBGATHERS = 2
IDX_PER_GATHER = (ROWS_PER_CHUNK // SUBGATHERS) * F
PAD_PER_GATHER = (ROWS_PER_CHUNK // SUBGATHERS) * FPAD
IDX_PER_CHUNK = ROWS_PER_CHUNK * F
PAD_PER_CHUNK = ROWS_PER_CHUNK * FPAD
RING = 8


def _fm_body(x_hbm, xp_hbm, w0_hbm, bias_hbm, emb_hbm, out_hbm,
             xv, xpv, w0v, ebs, bbs, outv, esem, bsem):
  wid = lax.axis_index("s") * NUM_CORES + lax.axis_index("c")
  emb2d = emb_hbm
  pltpu.sync_copy(
      x_hbm.at[pl.ds(pl.multiple_of(wid * BPW * F, 8), BPW * F)], xv)
  pltpu.sync_copy(
      xp_hbm.at[pl.ds(pl.multiple_of(wid * BPW * FPAD, 8), BPW * FPAD)], xpv)
  pltpu.sync_copy(w0_hbm, w0v)

  def copies(c, slot):
    for j in range(SUBGATHERS):
      eoff = pl.multiple_of(c * IDX_PER_CHUNK + j * IDX_PER_GATHER, 8)
      boff = pl.multiple_of(c * PAD_PER_CHUNK + j * PAD_PER_GATHER, 8)
      yield pltpu.make_async_copy(
          emb2d.at[xv.at[pl.ds(eoff, IDX_PER_GATHER)]],
          ebs.at[slot, pl.ds(j * IDX_PER_GATHER, IDX_PER_GATHER)],
          esem.at[slot])
      yield pltpu.make_async_copy(
          bias_hbm.at[xpv.at[pl.ds(boff, PAD_PER_GATHER)]],
          bbs.at[slot, pl.ds(j * PAD_PER_GATHER, PAD_PER_GATHER)],
          bsem.at[slot])

  def issue(c, slot):
    for cp in copies(c, slot):
      cp.start()

  def wait(c, slot):
    for cp in copies(c, slot):
      cp.wait()

  lane = lax.iota(jnp.int32, LANES)
  bias_mask = lane < (F - LANES)
  res_mask = lane < ROWS_PER_CHUNK

  def compute(c, slot):
    res = jnp.zeros((LANES,), jnp.float32)
    for r in range(ROWS_PER_CHUNK):
      acc0 = jnp.zeros((LANES,), jnp.float32)
      acc1 = jnp.zeros((LANES,), jnp.float32)
      sq0 = jnp.zeros((LANES,), jnp.float32)
      sq1 = jnp.zeros((LANES,), jnp.float32)
      for f in range(F):
        row = r * F + f
        v0 = ebs[slot, row, pl.ds(0, LANES)]
        v1 = ebs[slot, row, pl.ds(LANES, LANES)]
        acc0 = acc0 + v0
        sq0 = sq0 + v0 * v0
        acc1 = acc1 + v1
        sq1 = sq1 + v1 * v1
      t = acc0 * acc0 + acc1 * acc1 - sq0 - sq1
      b0 = bbs[slot, pl.ds(r * FPAD, LANES)]
      b1 = bbs[slot, pl.ds(r * FPAD + LANES, LANES)]
      u = 0.5 * t + b0 + jnp.where(bias_mask, b1, 0.0)
      total = jnp.sum(u)
      res = jnp.where(lane == r, total, res)
    res = res + w0v[...]
    plsc.store_compressed(outv.at[pl.ds(c * ROWS_PER_CHUNK, LANES)],
                          res, mask=res_mask)

  for c in range(RING):
    issue(c, c)

  @pl.loop(0, CHUNKS)
  def _chunk_loop(c):
    slot = lax.rem(c, RING)
    wait(c, slot)
    compute(c, slot)
    nxt = c + RING

    @pl.when(nxt < CHUNKS)
    def _():
      issue(nxt, slot)

  pltpu.sync_copy(outv.at[pl.ds(0, BPW)],
                  out_hbm.at[pl.ds(pl.multiple_of(wid * BPW, BPW), BPW)])


@jax.jit
def _fm_call(x_flat, xp_flat, w016, bias_flat, emb_wide):
  return pl.kernel(
      _fm_body,
      out_type=jax.ShapeDtypeStruct((B,), jnp.float32),
      mesh=plsc.VectorSubcoreMesh(core_axis_name="c", subcore_axis_name="s"),
      compiler_params=pltpu.CompilerParams(
          needs_layout_passes=False, use_tc_tiling_on_sc=False),
      scratch_types=[
          pltpu.VMEM((BPW * F,), jnp.int32),
          pltpu.VMEM((BPW * FPAD,), jnp.int32),
          pltpu.VMEM((LANES,), jnp.float32),
          pltpu.VMEM((RING, IDX_PER_CHUNK, D), jnp.float32),
          pltpu.VMEM((RING, PAD_PER_CHUNK), jnp.float32),
          pltpu.VMEM((BPW + LANES,), jnp.float32),
          pltpu.SemaphoreType.DMA((RING,)),
          pltpu.SemaphoreType.DMA((RING,)),
      ],
  )(x_flat, xp_flat, w016, bias_flat, emb_wide)


def kernel(x, w0, bias_table, emb_table):
  x = x.astype(jnp.int32)
  xpad = jnp.pad(x, ((0, 0), (0, FPAD - F)))
  w016 = jnp.broadcast_to(w0.astype(jnp.float32), (LANES,))
  return _fm_call(x.reshape(-1), xpad.reshape(-1), w016,
                  bias_table.reshape(-1), emb_table)
